# baseline (speedup 1.0000x reference)
import functools

import jax
import jax.numpy as jnp
from jax import lax
from jax.experimental import pallas as pl
from jax.experimental.pallas import tpu as pltpu
from jax.experimental.pallas import tpu_sc as plsc

_B = 4194304
_ROWS = 4096
_COLS = 1024
_BLK_R = 8
_GRID = _ROWS // _BLK_R

_NW = 32
_L = 16
_PER_W = _B // _NW
_CH = 16384
_NCH = _PER_W // _CH
_NB1 = 2048
_NB2 = 4096


def _tc_pass1(x0_ref, x1_ref, t_ref, v_ref, an_ref, ap_ref, al_ref):
    @pl.when(pl.program_id(0) == 0)
    def _init():
        an_ref[...] = jnp.zeros_like(an_ref)
        ap_ref[...] = jnp.zeros_like(ap_ref)
        al_ref[...] = jnp.zeros_like(al_ref)

    d = x0_ref[...] - x1_ref[...]
    tf = t_ref[...].astype(jnp.float32)
    sp_n = jnp.maximum(-d, 0.0) + jnp.log1p(jnp.exp(-jnp.abs(d)))
    sp_p = d + sp_n
    v = sp_n * (1.0 - tf)
    v_ref[...] = v

    def fold(z):
        acc = z[:, 0:128]
        for k in range(1, 8):
            acc = acc + z[:, 128 * k:128 * (k + 1)]
        return acc

    an_ref[...] += fold(tf)
    ap_ref[...] += fold(sp_p * tf)
    al_ref[...] += fold(v)


def _run_tc_pass1(x0, x1, t):
    return pl.pallas_call(
        _tc_pass1,
        grid=(_GRID,),
        in_specs=[
            pl.BlockSpec((_BLK_R, _COLS), lambda i: (i, 0)),
            pl.BlockSpec((_BLK_R, _COLS), lambda i: (i, 0)),
            pl.BlockSpec((_BLK_R, _COLS), lambda i: (i, 0)),
        ],
        out_specs=[
            pl.BlockSpec((_BLK_R, _COLS), lambda i: (i, 0)),
            pl.BlockSpec((8, 128), lambda i: (0, 0)),
            pl.BlockSpec((8, 128), lambda i: (0, 0)),
            pl.BlockSpec((8, 128), lambda i: (0, 0)),
        ],
        out_shape=[
            jax.ShapeDtypeStruct((_ROWS, _COLS), jnp.float32),
            jax.ShapeDtypeStruct((8, 128), jnp.float32),
            jax.ShapeDtypeStruct((8, 128), jnp.float32),
            jax.ShapeDtypeStruct((8, 128), jnp.float32),
        ],
    )(x0, x1, t)


def _sc_hist1_body(v_hbm, cnt_hbm, buf0, buf1, cnt_v, sem0, sem1):
    wid = lax.axis_index("s") * 2 + lax.axis_index("c")
    base = wid * _PER_W

    def zero_body(i, _):
        cnt_v[pl.ds(i * _L, _L)] = jnp.zeros((_L,), jnp.float32)
        return 0

    lax.fori_loop(0, _NB1 // _L, zero_body, 0)

    ones = jnp.ones((_L,), jnp.float32)
    bufs = (buf0, buf1)
    sems = (sem0, sem1)
    copies = [None, None]
    copies[0] = pltpu.async_copy(v_hbm.at[pl.ds(base, _CH)], buf0, sem0)
    for c in range(_NCH):
        if c + 1 < _NCH:
            copies[(c + 1) % 2] = pltpu.async_copy(
                v_hbm.at[pl.ds(base + (c + 1) * _CH, _CH)],
                bufs[(c + 1) % 2], sems[(c + 1) % 2])
        copies[c % 2].wait()
        buf = bufs[c % 2]

        def vec_body(i, _):
            v = buf[pl.ds(i * _L, _L)]
            bits = lax.bitcast_convert_type(v, jnp.int32)
            b = lax.shift_right_logical(bits, 21)
            plsc.addupdate_scatter(cnt_v, [b], ones)
            return 0

        lax.fori_loop(0, _CH // _L, vec_body, 0, unroll=8)
    pltpu.sync_copy(cnt_v, cnt_hbm.at[wid])


def _run_sc_hist1(v):
    mesh = plsc.VectorSubcoreMesh(core_axis_name="c", subcore_axis_name="s")
    kern = functools.partial(
        pl.kernel,
        mesh=mesh,
        out_type=jax.ShapeDtypeStruct((_NW, _NB1), jnp.float32),
        scratch_types=[
            pltpu.VMEM((_CH,), jnp.float32),
            pltpu.VMEM((_CH,), jnp.float32),
            pltpu.VMEM((_NB1,), jnp.float32),
            pltpu.SemaphoreType.DMA,
            pltpu.SemaphoreType.DMA,
        ],
        compiler_params=pltpu.CompilerParams(needs_layout_passes=False),
    )(_sc_hist1_body)
    return kern(v.reshape(_B))


def _suffix_2d(h2d, rows, cols):
    ri = lax.broadcasted_iota(jnp.int32, (rows, rows), 0)
    rj = lax.broadcasted_iota(jnp.int32, (rows, rows), 1)
    m_rows = (ri < rj).astype(jnp.float32)
    ci = lax.broadcasted_iota(jnp.int32, (cols, cols), 0)
    cj = lax.broadcasted_iota(jnp.int32, (cols, cols), 1)
    m_cols = (ci > cj).astype(jnp.float32)
    rs = jnp.sum(h2d, axis=1, keepdims=True)
    rows_after = jax.lax.dot_general(
        m_rows, rs, (((1,), (0,)), ((), ())),
        preferred_element_type=jnp.float32,
        precision=lax.Precision.HIGHEST)
    within = jax.lax.dot_general(
        h2d, m_cols, (((1,), (0,)), ((), ())),
        preferred_element_type=jnp.float32,
        precision=lax.Precision.HIGHEST)
    return rows_after + within


def _select_bucket(cnt2d, rank, rows, cols):
    g = _suffix_2d(cnt2d, rows, cols)
    m = jnp.logical_and(g < rank, g + cnt2d >= rank).astype(jnp.float32)
    ri = lax.broadcasted_iota(jnp.int32, (rows, cols), 0)
    ci = lax.broadcasted_iota(jnp.int32, (rows, cols), 1)
    idx = jnp.sum((ri * cols + ci).astype(jnp.float32) * m)
    r_in = jnp.sum(m * (rank - g))
    return idx, r_in


def _tc_decide1(cnt_ref, an_ref, j_ref, r_ref):
    cnt2d = jnp.sum(cnt_ref[...], axis=0)
    n = jnp.sum(an_ref[...])
    idx, r_in = _select_bucket(cnt2d, n, 16, 128)
    j_ref[...] = jnp.full((8, 128), idx.astype(jnp.int32), jnp.int32)
    r_ref[...] = jnp.full((8, 128), r_in, jnp.float32)


def _run_tc_decide1(cnt1, an):
    return pl.pallas_call(
        _tc_decide1,
        out_shape=[
            jax.ShapeDtypeStruct((8, 128), jnp.int32),
            jax.ShapeDtypeStruct((8, 128), jnp.float32),
        ],
    )(cnt1.reshape(_NW, 16, 128), an)


def _sc_hist2_body(v_hbm, j_hbm, cnt_hbm, buf0, buf1, jbuf, cnt_v,
                   sem0, sem1):
    wid = lax.axis_index("s") * 2 + lax.axis_index("c")
    base = wid * _PER_W

    def zero_body(i, _):
        cnt_v[pl.ds(i * _L, _L)] = jnp.zeros((_L,), jnp.float32)
        return 0

    lax.fori_loop(0, _NB2 // _L, zero_body, 0)

    pltpu.sync_copy(j_hbm, jbuf)
    jv = jbuf[...]
    ones = jnp.ones((_L,), jnp.float32)
    bufs = (buf0, buf1)
    sems = (sem0, sem1)
    copies = [None, None]
    copies[0] = pltpu.async_copy(v_hbm.at[pl.ds(base, _CH)], buf0, sem0)
    for c in range(_NCH):
        if c + 1 < _NCH:
            copies[(c + 1) % 2] = pltpu.async_copy(
                v_hbm.at[pl.ds(base + (c + 1) * _CH, _CH)],
                bufs[(c + 1) % 2], sems[(c + 1) % 2])
        copies[c % 2].wait()
        buf = bufs[c % 2]

        def vec_body(i, _):
            v = buf[pl.ds(i * _L, _L)]
            bits = lax.bitcast_convert_type(v, jnp.int32)
            hi = lax.shift_right_logical(bits, 21)
            msk = hi == jv
            sub = jnp.bitwise_and(lax.shift_right_logical(bits, 9), _NB2 - 1)
            plsc.addupdate_scatter(cnt_v, [sub], ones, mask=msk)
            return 0

        lax.fori_loop(0, _CH // _L, vec_body, 0, unroll=8)
    pltpu.sync_copy(cnt_v, cnt_hbm.at[wid])


def _run_sc_hist2(v, jvec):
    mesh = plsc.VectorSubcoreMesh(core_axis_name="c", subcore_axis_name="s")
    kern = functools.partial(
        pl.kernel,
        mesh=mesh,
        out_type=jax.ShapeDtypeStruct((_NW, _NB2), jnp.float32),
        scratch_types=[
            pltpu.VMEM((_CH,), jnp.float32),
            pltpu.VMEM((_CH,), jnp.float32),
            pltpu.VMEM((_L,), jnp.int32),
            pltpu.VMEM((_NB2,), jnp.float32),
            pltpu.SemaphoreType.DMA,
            pltpu.SemaphoreType.DMA,
        ],
        compiler_params=pltpu.CompilerParams(needs_layout_passes=False),
    )(_sc_hist2_body)
    return kern(v.reshape(_B), jvec)


def _tc_decide2(cnt_ref, j_ref, r_ref, t_ref):
    cnt2d = jnp.sum(cnt_ref[...], axis=0)
    rank = r_ref[0, 0]
    idx, _ = _select_bucket(cnt2d, rank, 32, 128)
    jstar = j_ref[0, 0]
    tbits = jnp.bitwise_or(lax.shift_left(jstar, 21),
                           lax.shift_left(idx.astype(jnp.int32), 9))
    tval = lax.bitcast_convert_type(tbits, jnp.float32)
    t_ref[...] = jnp.full((8, 128), tval, jnp.float32)


def _run_tc_decide2(cnt2, jout, rout):
    return pl.pallas_call(
        _tc_decide2,
        out_shape=jax.ShapeDtypeStruct((8, 128), jnp.float32),
    )(cnt2.reshape(_NW, 32, 128), jout, rout)


def _tc_final(v_ref, t_ref, an_ref, ap_ref, al_ref, out_ref, cs_ref, cc_ref):
    @pl.when(pl.program_id(0) == 0)
    def _init():
        cs_ref[...] = jnp.zeros_like(cs_ref)
        cc_ref[...] = jnp.zeros_like(cc_ref)

    tval = t_ref[0, 0]
    v = v_ref[...]
    gt = (v > tval).astype(jnp.float32)

    def fold(z):
        acc = z[:, 0:128]
        for k in range(1, 8):
            acc = acc + z[:, 128 * k:128 * (k + 1)]
        return acc

    cs_ref[...] += fold(v * gt)
    cc_ref[...] += fold(gt)

    @pl.when(pl.program_id(0) == _GRID - 1)
    def _finish():
        n = jnp.sum(an_ref[...])
        loss_pos = jnp.sum(ap_ref[...])
        loss_all_neg = jnp.sum(al_ref[...])
        sum_gt = jnp.sum(cs_ref[...])
        cnt_gt = jnp.sum(cc_ref[...])
        t_sum = sum_gt + (n - cnt_gt) * tval
        ce = (loss_pos + loss_all_neg) / jnp.float32(_B)
        res = (loss_pos + t_sum) / (2.0 * n) + ce
        out_ref[...] = jnp.full((1, 1), res, jnp.float32)


def _run_tc_final(v, tout, an, ap, al):
    return pl.pallas_call(
        _tc_final,
        grid=(_GRID,),
        in_specs=[
            pl.BlockSpec((_BLK_R, _COLS), lambda i: (i, 0)),
            pl.BlockSpec((8, 128), lambda i: (0, 0)),
            pl.BlockSpec((8, 128), lambda i: (0, 0)),
            pl.BlockSpec((8, 128), lambda i: (0, 0)),
            pl.BlockSpec((8, 128), lambda i: (0, 0)),
        ],
        out_specs=pl.BlockSpec((1, 1), lambda i: (0, 0)),
        out_shape=jax.ShapeDtypeStruct((1, 1), jnp.float32),
        scratch_shapes=[
            pltpu.VMEM((8, 128), jnp.float32),
            pltpu.VMEM((8, 128), jnp.float32),
        ],
    )(v, tout, an, ap, al)


def kernel(input, target):
    x0 = input[:, 0].reshape(_ROWS, _COLS)
    x1 = input[:, 1].reshape(_ROWS, _COLS)
    t = target.reshape(_ROWS, _COLS)
    v, an, ap, al = _run_tc_pass1(x0, x1, t)
    cnt1 = _run_sc_hist1(v)
    jout, rout = _run_tc_decide1(cnt1, an)
    jvec = jout[0, :_L].reshape(_L)
    cnt2 = _run_sc_hist2(v, jvec)
    tout = _run_tc_decide2(cnt2, jout, rout)
    out = _run_tc_final(v, tout, an, ap, al)
    return out.reshape(())


# speedup vs baseline: 1.7366x; 1.7366x over previous
import functools

import jax
import jax.numpy as jnp
from jax import lax
from jax.experimental import pallas as pl
from jax.experimental.pallas import tpu as pltpu
from jax.experimental.pallas import tpu_sc as plsc

_B = 4194304
_ROWS = 4096
_COLS = 1024
_BLK_R = 64
_GRID = _ROWS // _BLK_R

_NW = 32
_L = 16
_PER_W = _B // _NW
_CH = 16384
_NCH = _PER_W // _CH
_NB1 = 2048
_NB2 = 4096


def _tc_pass1(x0_ref, x1_ref, t_ref, v_ref, an_ref, ap_ref, al_ref):
    @pl.when(pl.program_id(0) == 0)
    def _init():
        an_ref[...] = jnp.zeros_like(an_ref)
        ap_ref[...] = jnp.zeros_like(ap_ref)
        al_ref[...] = jnp.zeros_like(al_ref)

    d = x0_ref[...] - x1_ref[...]
    tf = t_ref[...].astype(jnp.float32)
    sp_n = jnp.maximum(-d, 0.0) + jnp.log1p(jnp.exp(-jnp.abs(d)))
    sp_p = d + sp_n
    v = sp_n * (1.0 - tf)
    v_ref[...] = v

    def fold(z):
        acc = z[:, 0:128]
        for k in range(1, 8):
            acc = acc + z[:, 128 * k:128 * (k + 1)]
        acc8 = acc[0:8, :]
        for k in range(1, _BLK_R // 8):
            acc8 = acc8 + acc[8 * k:8 * (k + 1), :]
        return acc8

    an_ref[...] += fold(tf)
    ap_ref[...] += fold(sp_p * tf)
    al_ref[...] += fold(v)


def _run_tc_pass1(x0, x1, t):
    return pl.pallas_call(
        _tc_pass1,
        grid=(_GRID,),
        in_specs=[
            pl.BlockSpec((_BLK_R, _COLS), lambda i: (i, 0)),
            pl.BlockSpec((_BLK_R, _COLS), lambda i: (i, 0)),
            pl.BlockSpec((_BLK_R, _COLS), lambda i: (i, 0)),
        ],
        out_specs=[
            pl.BlockSpec((_BLK_R, _COLS), lambda i: (i, 0)),
            pl.BlockSpec((8, 128), lambda i: (0, 0)),
            pl.BlockSpec((8, 128), lambda i: (0, 0)),
            pl.BlockSpec((8, 128), lambda i: (0, 0)),
        ],
        out_shape=[
            jax.ShapeDtypeStruct((_ROWS, _COLS), jnp.float32),
            jax.ShapeDtypeStruct((8, 128), jnp.float32),
            jax.ShapeDtypeStruct((8, 128), jnp.float32),
            jax.ShapeDtypeStruct((8, 128), jnp.float32),
        ],
    )(x0, x1, t)


def _sc_hist1_body(v_hbm, cnt_hbm, buf0, buf1, cnt_v, sem0, sem1):
    wid = lax.axis_index("s") * 2 + lax.axis_index("c")
    base = wid * _PER_W

    def zero_body(i, _):
        cnt_v[pl.ds(i * _L, _L)] = jnp.zeros((_L,), jnp.float32)
        return 0

    lax.fori_loop(0, _NB1 // _L, zero_body, 0)

    ones = jnp.ones((_L,), jnp.float32)
    bufs = (buf0, buf1)
    sems = (sem0, sem1)
    copies = [None, None]
    copies[0] = pltpu.async_copy(v_hbm.at[pl.ds(base, _CH)], buf0, sem0)
    for c in range(_NCH):
        if c + 1 < _NCH:
            copies[(c + 1) % 2] = pltpu.async_copy(
                v_hbm.at[pl.ds(base + (c + 1) * _CH, _CH)],
                bufs[(c + 1) % 2], sems[(c + 1) % 2])
        copies[c % 2].wait()
        buf = bufs[c % 2]

        def vec_body(i, _):
            v = buf[pl.ds(i * _L, _L)]
            bits = lax.bitcast_convert_type(v, jnp.int32)
            b = lax.shift_right_logical(bits, 21)
            plsc.addupdate_scatter(cnt_v, [b], ones)
            return 0

        lax.fori_loop(0, _CH // _L, vec_body, 0, unroll=8)
    pltpu.sync_copy(cnt_v, cnt_hbm.at[wid])


def _run_sc_hist1(v):
    mesh = plsc.VectorSubcoreMesh(core_axis_name="c", subcore_axis_name="s")
    kern = functools.partial(
        pl.kernel,
        mesh=mesh,
        out_type=jax.ShapeDtypeStruct((_NW, _NB1), jnp.float32),
        scratch_types=[
            pltpu.VMEM((_CH,), jnp.float32),
            pltpu.VMEM((_CH,), jnp.float32),
            pltpu.VMEM((_NB1,), jnp.float32),
            pltpu.SemaphoreType.DMA,
            pltpu.SemaphoreType.DMA,
        ],
        compiler_params=pltpu.CompilerParams(needs_layout_passes=False),
    )(_sc_hist1_body)
    return kern(v.reshape(_B))


def _suffix_2d(h2d, rows, cols):
    ri = lax.broadcasted_iota(jnp.int32, (rows, rows), 0)
    rj = lax.broadcasted_iota(jnp.int32, (rows, rows), 1)
    m_rows = (ri < rj).astype(jnp.float32)
    ci = lax.broadcasted_iota(jnp.int32, (cols, cols), 0)
    cj = lax.broadcasted_iota(jnp.int32, (cols, cols), 1)
    m_cols = (ci > cj).astype(jnp.float32)
    rs = jnp.sum(h2d, axis=1, keepdims=True)
    rows_after = jax.lax.dot_general(
        m_rows, rs, (((1,), (0,)), ((), ())),
        preferred_element_type=jnp.float32,
        precision=lax.Precision.HIGHEST)
    within = jax.lax.dot_general(
        h2d, m_cols, (((1,), (0,)), ((), ())),
        preferred_element_type=jnp.float32,
        precision=lax.Precision.HIGHEST)
    return rows_after + within


def _select_bucket(cnt2d, rank, rows, cols):
    g = _suffix_2d(cnt2d, rows, cols)
    m = jnp.logical_and(g < rank, g + cnt2d >= rank).astype(jnp.float32)
    ri = lax.broadcasted_iota(jnp.int32, (rows, cols), 0)
    ci = lax.broadcasted_iota(jnp.int32, (rows, cols), 1)
    idx = jnp.sum((ri * cols + ci).astype(jnp.float32) * m)
    r_in = jnp.sum(m * (rank - g))
    return idx, r_in


def _tc_decide1(cnt_ref, an_ref, j_ref, r_ref):
    cnt2d = jnp.sum(cnt_ref[...], axis=0)
    n = jnp.sum(an_ref[...])
    idx, r_in = _select_bucket(cnt2d, n, 16, 128)
    j_ref[...] = jnp.full((8, 128), idx.astype(jnp.int32), jnp.int32)
    r_ref[...] = jnp.full((8, 128), r_in, jnp.float32)


def _run_tc_decide1(cnt1, an):
    return pl.pallas_call(
        _tc_decide1,
        out_shape=[
            jax.ShapeDtypeStruct((8, 128), jnp.int32),
            jax.ShapeDtypeStruct((8, 128), jnp.float32),
        ],
    )(cnt1.reshape(_NW, 16, 128), an)


def _sc_hist2_body(v_hbm, j_hbm, cnt_hbm, buf0, buf1, jbuf, cnt_v,
                   sem0, sem1):
    wid = lax.axis_index("s") * 2 + lax.axis_index("c")
    base = wid * _PER_W

    def zero_body(i, _):
        cnt_v[pl.ds(i * _L, _L)] = jnp.zeros((_L,), jnp.float32)
        return 0

    lax.fori_loop(0, _NB2 // _L, zero_body, 0)

    pltpu.sync_copy(j_hbm, jbuf)
    jv = jbuf[...]
    ones = jnp.ones((_L,), jnp.float32)
    bufs = (buf0, buf1)
    sems = (sem0, sem1)
    copies = [None, None]
    copies[0] = pltpu.async_copy(v_hbm.at[pl.ds(base, _CH)], buf0, sem0)
    for c in range(_NCH):
        if c + 1 < _NCH:
            copies[(c + 1) % 2] = pltpu.async_copy(
                v_hbm.at[pl.ds(base + (c + 1) * _CH, _CH)],
                bufs[(c + 1) % 2], sems[(c + 1) % 2])
        copies[c % 2].wait()
        buf = bufs[c % 2]

        def vec_body(i, _):
            v = buf[pl.ds(i * _L, _L)]
            bits = lax.bitcast_convert_type(v, jnp.int32)
            hi = lax.shift_right_logical(bits, 21)
            msk = hi == jv
            sub = jnp.bitwise_and(lax.shift_right_logical(bits, 9), _NB2 - 1)
            plsc.addupdate_scatter(cnt_v, [sub], ones, mask=msk)
            return 0

        lax.fori_loop(0, _CH // _L, vec_body, 0, unroll=8)
    pltpu.sync_copy(cnt_v, cnt_hbm.at[wid])


def _run_sc_hist2(v, jvec):
    mesh = plsc.VectorSubcoreMesh(core_axis_name="c", subcore_axis_name="s")
    kern = functools.partial(
        pl.kernel,
        mesh=mesh,
        out_type=jax.ShapeDtypeStruct((_NW, _NB2), jnp.float32),
        scratch_types=[
            pltpu.VMEM((_CH,), jnp.float32),
            pltpu.VMEM((_CH,), jnp.float32),
            pltpu.VMEM((_L,), jnp.int32),
            pltpu.VMEM((_NB2,), jnp.float32),
            pltpu.SemaphoreType.DMA,
            pltpu.SemaphoreType.DMA,
        ],
        compiler_params=pltpu.CompilerParams(needs_layout_passes=False),
    )(_sc_hist2_body)
    return kern(v.reshape(_B), jvec)


def _tc_decide2(cnt_ref, j_ref, r_ref, t_ref):
    cnt2d = jnp.sum(cnt_ref[...], axis=0)
    rank = r_ref[0, 0]
    idx, _ = _select_bucket(cnt2d, rank, 32, 128)
    jstar = j_ref[0, 0]
    tbits = jnp.bitwise_or(lax.shift_left(jstar, 21),
                           lax.shift_left(idx.astype(jnp.int32), 9))
    tval = lax.bitcast_convert_type(tbits, jnp.float32)
    t_ref[...] = jnp.full((8, 128), tval, jnp.float32)


def _run_tc_decide2(cnt2, jout, rout):
    return pl.pallas_call(
        _tc_decide2,
        out_shape=jax.ShapeDtypeStruct((8, 128), jnp.float32),
    )(cnt2.reshape(_NW, 32, 128), jout, rout)


def _tc_final(v_ref, t_ref, an_ref, ap_ref, al_ref, out_ref, cs_ref, cc_ref):
    @pl.when(pl.program_id(0) == 0)
    def _init():
        cs_ref[...] = jnp.zeros_like(cs_ref)
        cc_ref[...] = jnp.zeros_like(cc_ref)

    tval = t_ref[0, 0]
    v = v_ref[...]
    gt = (v > tval).astype(jnp.float32)

    def fold(z):
        acc = z[:, 0:128]
        for k in range(1, 8):
            acc = acc + z[:, 128 * k:128 * (k + 1)]
        acc8 = acc[0:8, :]
        for k in range(1, _BLK_R // 8):
            acc8 = acc8 + acc[8 * k:8 * (k + 1), :]
        return acc8

    cs_ref[...] += fold(v * gt)
    cc_ref[...] += fold(gt)

    @pl.when(pl.program_id(0) == _GRID - 1)
    def _finish():
        n = jnp.sum(an_ref[...])
        loss_pos = jnp.sum(ap_ref[...])
        loss_all_neg = jnp.sum(al_ref[...])
        sum_gt = jnp.sum(cs_ref[...])
        cnt_gt = jnp.sum(cc_ref[...])
        t_sum = sum_gt + (n - cnt_gt) * tval
        ce = (loss_pos + loss_all_neg) / jnp.float32(_B)
        res = (loss_pos + t_sum) / (2.0 * n) + ce
        out_ref[...] = jnp.full((1, 1), res, jnp.float32)


def _run_tc_final(v, tout, an, ap, al):
    return pl.pallas_call(
        _tc_final,
        grid=(_GRID,),
        in_specs=[
            pl.BlockSpec((_BLK_R, _COLS), lambda i: (i, 0)),
            pl.BlockSpec((8, 128), lambda i: (0, 0)),
            pl.BlockSpec((8, 128), lambda i: (0, 0)),
            pl.BlockSpec((8, 128), lambda i: (0, 0)),
            pl.BlockSpec((8, 128), lambda i: (0, 0)),
        ],
        out_specs=pl.BlockSpec((1, 1), lambda i: (0, 0)),
        out_shape=jax.ShapeDtypeStruct((1, 1), jnp.float32),
        scratch_shapes=[
            pltpu.VMEM((8, 128), jnp.float32),
            pltpu.VMEM((8, 128), jnp.float32),
        ],
    )(v, tout, an, ap, al)


def kernel(input, target):
    x0 = input[:, 0].reshape(_ROWS, _COLS)
    x1 = input[:, 1].reshape(_ROWS, _COLS)
    t = target.reshape(_ROWS, _COLS)
    v, an, ap, al = _run_tc_pass1(x0, x1, t)
    cnt1 = _run_sc_hist1(v)
    jout, rout = _run_tc_decide1(cnt1, an)
    jvec = jout[0, :_L].reshape(_L)
    cnt2 = _run_sc_hist2(v, jvec)
    tout = _run_tc_decide2(cnt2, jout, rout)
    out = _run_tc_final(v, tout, an, ap, al)
    return out.reshape(())


# speedup vs baseline: 1.8261x; 1.0516x over previous
import functools

import jax
import jax.numpy as jnp
from jax import lax
from jax.experimental import pallas as pl
from jax.experimental.pallas import tpu as pltpu
from jax.experimental.pallas import tpu_sc as plsc

_B = 4194304
_ROWS = 4096
_COLS = 1024
_BLK_R = 128
_GRID = _ROWS // _BLK_R

_NW = 32
_L = 16
_PER_W = _B // _NW
_CH = 32768
_NCH = _PER_W // _CH
_NB1 = 2048
_NB2 = 4096


def _tc_pass1(x0_ref, x1_ref, t_ref, v_ref, an_ref, ap_ref, al_ref):
    @pl.when(pl.program_id(0) == 0)
    def _init():
        an_ref[...] = jnp.zeros_like(an_ref)
        ap_ref[...] = jnp.zeros_like(ap_ref)
        al_ref[...] = jnp.zeros_like(al_ref)

    d = x0_ref[...] - x1_ref[...]
    tf = t_ref[...].astype(jnp.float32)
    sp_n = jnp.maximum(-d, 0.0) + jnp.log1p(jnp.exp(-jnp.abs(d)))
    sp_p = d + sp_n
    v = sp_n * (1.0 - tf)
    v_ref[...] = v

    def fold(z):
        acc = z[:, 0:128]
        for k in range(1, 8):
            acc = acc + z[:, 128 * k:128 * (k + 1)]
        acc8 = acc[0:8, :]
        for k in range(1, _BLK_R // 8):
            acc8 = acc8 + acc[8 * k:8 * (k + 1), :]
        return acc8

    an_ref[...] += fold(tf)
    ap_ref[...] += fold(sp_p * tf)
    al_ref[...] += fold(v)


def _run_tc_pass1(x0, x1, t):
    return pl.pallas_call(
        _tc_pass1,
        grid=(_GRID,),
        in_specs=[
            pl.BlockSpec((_BLK_R, _COLS), lambda i: (i, 0)),
            pl.BlockSpec((_BLK_R, _COLS), lambda i: (i, 0)),
            pl.BlockSpec((_BLK_R, _COLS), lambda i: (i, 0)),
        ],
        out_specs=[
            pl.BlockSpec((_BLK_R, _COLS), lambda i: (i, 0)),
            pl.BlockSpec((8, 128), lambda i: (0, 0)),
            pl.BlockSpec((8, 128), lambda i: (0, 0)),
            pl.BlockSpec((8, 128), lambda i: (0, 0)),
        ],
        out_shape=[
            jax.ShapeDtypeStruct((_ROWS, _COLS), jnp.float32),
            jax.ShapeDtypeStruct((8, 128), jnp.float32),
            jax.ShapeDtypeStruct((8, 128), jnp.float32),
            jax.ShapeDtypeStruct((8, 128), jnp.float32),
        ],
    )(x0, x1, t)


def _sc_hist1_body(v_hbm, cnt_hbm, buf0, buf1, cnt_v, sem0, sem1):
    wid = lax.axis_index("s") * 2 + lax.axis_index("c")
    base = wid * _PER_W

    def zero_body(i, _):
        cnt_v[pl.ds(i * _L, _L)] = jnp.zeros((_L,), jnp.float32)
        return 0

    lax.fori_loop(0, _NB1 // _L, zero_body, 0)

    ones = jnp.ones((_L,), jnp.float32)
    bufs = (buf0, buf1)
    sems = (sem0, sem1)
    copies = [None, None]
    copies[0] = pltpu.async_copy(v_hbm.at[pl.ds(base, _CH)], buf0, sem0)
    for c in range(_NCH):
        if c + 1 < _NCH:
            copies[(c + 1) % 2] = pltpu.async_copy(
                v_hbm.at[pl.ds(base + (c + 1) * _CH, _CH)],
                bufs[(c + 1) % 2], sems[(c + 1) % 2])
        copies[c % 2].wait()
        buf = bufs[c % 2]

        def vec_body(i, _):
            v = buf[pl.ds(i * _L, _L)]
            bits = lax.bitcast_convert_type(v, jnp.int32)
            b = lax.shift_right_logical(bits, 21)
            plsc.addupdate_scatter(cnt_v, [b], ones)
            return 0

        lax.fori_loop(0, _CH // _L, vec_body, 0, unroll=8)
    pltpu.sync_copy(cnt_v, cnt_hbm.at[wid])


def _run_sc_hist1(v):
    mesh = plsc.VectorSubcoreMesh(core_axis_name="c", subcore_axis_name="s")
    kern = functools.partial(
        pl.kernel,
        mesh=mesh,
        out_type=jax.ShapeDtypeStruct((_NW, _NB1), jnp.float32),
        scratch_types=[
            pltpu.VMEM((_CH,), jnp.float32),
            pltpu.VMEM((_CH,), jnp.float32),
            pltpu.VMEM((_NB1,), jnp.float32),
            pltpu.SemaphoreType.DMA,
            pltpu.SemaphoreType.DMA,
        ],
        compiler_params=pltpu.CompilerParams(needs_layout_passes=False),
    )(_sc_hist1_body)
    return kern(v.reshape(_B))


def _suffix_2d(h2d, rows, cols):
    ri = lax.broadcasted_iota(jnp.int32, (rows, rows), 0)
    rj = lax.broadcasted_iota(jnp.int32, (rows, rows), 1)
    m_rows = (ri < rj).astype(jnp.float32)
    ci = lax.broadcasted_iota(jnp.int32, (cols, cols), 0)
    cj = lax.broadcasted_iota(jnp.int32, (cols, cols), 1)
    m_cols = (ci > cj).astype(jnp.float32)
    rs = jnp.sum(h2d, axis=1, keepdims=True)
    rows_after = jax.lax.dot_general(
        m_rows, rs, (((1,), (0,)), ((), ())),
        preferred_element_type=jnp.float32,
        precision=lax.Precision.HIGHEST)
    within = jax.lax.dot_general(
        h2d, m_cols, (((1,), (0,)), ((), ())),
        preferred_element_type=jnp.float32,
        precision=lax.Precision.HIGHEST)
    return rows_after + within


def _select_bucket(cnt2d, rank, rows, cols):
    g = _suffix_2d(cnt2d, rows, cols)
    m = jnp.logical_and(g < rank, g + cnt2d >= rank).astype(jnp.float32)
    ri = lax.broadcasted_iota(jnp.int32, (rows, cols), 0)
    ci = lax.broadcasted_iota(jnp.int32, (rows, cols), 1)
    idx = jnp.sum((ri * cols + ci).astype(jnp.float32) * m)
    r_in = jnp.sum(m * (rank - g))
    return idx, r_in


def _tc_decide1(cnt_ref, an_ref, j_ref, r_ref):
    cnt2d = jnp.sum(cnt_ref[...], axis=0)
    n = jnp.sum(an_ref[...])
    idx, r_in = _select_bucket(cnt2d, n, 16, 128)
    j_ref[...] = jnp.full((8, 128), idx.astype(jnp.int32), jnp.int32)
    r_ref[...] = jnp.full((8, 128), r_in, jnp.float32)


def _run_tc_decide1(cnt1, an):
    return pl.pallas_call(
        _tc_decide1,
        out_shape=[
            jax.ShapeDtypeStruct((8, 128), jnp.int32),
            jax.ShapeDtypeStruct((8, 128), jnp.float32),
        ],
    )(cnt1.reshape(_NW, 16, 128), an)


def _sc_hist2_body(v_hbm, j_hbm, cnt_hbm, buf0, buf1, jbuf, cnt_v,
                   sem0, sem1):
    wid = lax.axis_index("s") * 2 + lax.axis_index("c")
    base = wid * _PER_W

    def zero_body(i, _):
        cnt_v[pl.ds(i * _L, _L)] = jnp.zeros((_L,), jnp.float32)
        return 0

    lax.fori_loop(0, _NB2 // _L, zero_body, 0)

    pltpu.sync_copy(j_hbm, jbuf)
    jv = jbuf[...]
    ones = jnp.ones((_L,), jnp.float32)
    bufs = (buf0, buf1)
    sems = (sem0, sem1)
    copies = [None, None]
    copies[0] = pltpu.async_copy(v_hbm.at[pl.ds(base, _CH)], buf0, sem0)
    for c in range(_NCH):
        if c + 1 < _NCH:
            copies[(c + 1) % 2] = pltpu.async_copy(
                v_hbm.at[pl.ds(base + (c + 1) * _CH, _CH)],
                bufs[(c + 1) % 2], sems[(c + 1) % 2])
        copies[c % 2].wait()
        buf = bufs[c % 2]

        def vec_body(i, _):
            v = buf[pl.ds(i * _L, _L)]
            bits = lax.bitcast_convert_type(v, jnp.int32)
            hi = lax.shift_right_logical(bits, 21)
            msk = hi == jv
            sub = jnp.bitwise_and(lax.shift_right_logical(bits, 9), _NB2 - 1)
            plsc.addupdate_scatter(cnt_v, [sub], ones, mask=msk)
            return 0

        lax.fori_loop(0, _CH // _L, vec_body, 0, unroll=8)
    pltpu.sync_copy(cnt_v, cnt_hbm.at[wid])


def _run_sc_hist2(v, jvec):
    mesh = plsc.VectorSubcoreMesh(core_axis_name="c", subcore_axis_name="s")
    kern = functools.partial(
        pl.kernel,
        mesh=mesh,
        out_type=jax.ShapeDtypeStruct((_NW, _NB2), jnp.float32),
        scratch_types=[
            pltpu.VMEM((_CH,), jnp.float32),
            pltpu.VMEM((_CH,), jnp.float32),
            pltpu.VMEM((_L,), jnp.int32),
            pltpu.VMEM((_NB2,), jnp.float32),
            pltpu.SemaphoreType.DMA,
            pltpu.SemaphoreType.DMA,
        ],
        compiler_params=pltpu.CompilerParams(needs_layout_passes=False),
    )(_sc_hist2_body)
    return kern(v.reshape(_B), jvec)


def _tc_decide2(cnt_ref, j_ref, r_ref, t_ref):
    cnt2d = jnp.sum(cnt_ref[...], axis=0)
    rank = r_ref[0, 0]
    idx, _ = _select_bucket(cnt2d, rank, 32, 128)
    jstar = j_ref[0, 0]
    tbits = jnp.bitwise_or(lax.shift_left(jstar, 21),
                           lax.shift_left(idx.astype(jnp.int32), 9))
    tval = lax.bitcast_convert_type(tbits, jnp.float32)
    t_ref[...] = jnp.full((8, 128), tval, jnp.float32)


def _run_tc_decide2(cnt2, jout, rout):
    return pl.pallas_call(
        _tc_decide2,
        out_shape=jax.ShapeDtypeStruct((8, 128), jnp.float32),
    )(cnt2.reshape(_NW, 32, 128), jout, rout)


def _tc_final(v_ref, t_ref, an_ref, ap_ref, al_ref, out_ref, cs_ref, cc_ref):
    @pl.when(pl.program_id(0) == 0)
    def _init():
        cs_ref[...] = jnp.zeros_like(cs_ref)
        cc_ref[...] = jnp.zeros_like(cc_ref)

    tval = t_ref[0, 0]
    v = v_ref[...]
    gt = (v > tval).astype(jnp.float32)

    def fold(z):
        acc = z[:, 0:128]
        for k in range(1, 8):
            acc = acc + z[:, 128 * k:128 * (k + 1)]
        acc8 = acc[0:8, :]
        for k in range(1, _BLK_R // 8):
            acc8 = acc8 + acc[8 * k:8 * (k + 1), :]
        return acc8

    cs_ref[...] += fold(v * gt)
    cc_ref[...] += fold(gt)

    @pl.when(pl.program_id(0) == _GRID - 1)
    def _finish():
        n = jnp.sum(an_ref[...])
        loss_pos = jnp.sum(ap_ref[...])
        loss_all_neg = jnp.sum(al_ref[...])
        sum_gt = jnp.sum(cs_ref[...])
        cnt_gt = jnp.sum(cc_ref[...])
        t_sum = sum_gt + (n - cnt_gt) * tval
        ce = (loss_pos + loss_all_neg) / jnp.float32(_B)
        res = (loss_pos + t_sum) / (2.0 * n) + ce
        out_ref[...] = jnp.full((1, 1), res, jnp.float32)


def _run_tc_final(v, tout, an, ap, al):
    return pl.pallas_call(
        _tc_final,
        grid=(_GRID,),
        in_specs=[
            pl.BlockSpec((_BLK_R, _COLS), lambda i: (i, 0)),
            pl.BlockSpec((8, 128), lambda i: (0, 0)),
            pl.BlockSpec((8, 128), lambda i: (0, 0)),
            pl.BlockSpec((8, 128), lambda i: (0, 0)),
            pl.BlockSpec((8, 128), lambda i: (0, 0)),
        ],
        out_specs=pl.BlockSpec((1, 1), lambda i: (0, 0)),
        out_shape=jax.ShapeDtypeStruct((1, 1), jnp.float32),
        scratch_shapes=[
            pltpu.VMEM((8, 128), jnp.float32),
            pltpu.VMEM((8, 128), jnp.float32),
        ],
    )(v, tout, an, ap, al)


def kernel(input, target):
    x0 = input[:, 0].reshape(_ROWS, _COLS)
    x1 = input[:, 1].reshape(_ROWS, _COLS)
    t = target.reshape(_ROWS, _COLS)
    v, an, ap, al = _run_tc_pass1(x0, x1, t)
    cnt1 = _run_sc_hist1(v)
    jout, rout = _run_tc_decide1(cnt1, an)
    jvec = jout[0, :_L].reshape(_L)
    cnt2 = _run_sc_hist2(v, jvec)
    tout = _run_tc_decide2(cnt2, jout, rout)
    out = _run_tc_final(v, tout, an, ap, al)
    return out.reshape(())


# speedup vs baseline: 1.8822x; 1.0307x over previous
import functools

import jax
import jax.numpy as jnp
from jax import lax
from jax.experimental import pallas as pl
from jax.experimental.pallas import tpu as pltpu
from jax.experimental.pallas import tpu_sc as plsc

_B = 4194304
_ROWS = 4096
_COLS = 1024
_BLK_R = 256
_GRID = _ROWS // _BLK_R

_NW = 32
_L = 16
_PER_W = _B // _NW
_CH = 32768
_NCH = _PER_W // _CH
_NB1 = 2048
_NB2 = 4096


def _tc_pass1(x0_ref, x1_ref, t_ref, v_ref, an_ref, ap_ref, al_ref):
    @pl.when(pl.program_id(0) == 0)
    def _init():
        an_ref[...] = jnp.zeros_like(an_ref)
        ap_ref[...] = jnp.zeros_like(ap_ref)
        al_ref[...] = jnp.zeros_like(al_ref)

    d = x0_ref[...] - x1_ref[...]
    tf = t_ref[...].astype(jnp.float32)
    sp_n = jnp.maximum(-d, 0.0) + jnp.log1p(jnp.exp(-jnp.abs(d)))
    sp_p = d + sp_n
    v = sp_n * (1.0 - tf)
    v_ref[...] = v

    def fold(z):
        acc = z[:, 0:128]
        for k in range(1, 8):
            acc = acc + z[:, 128 * k:128 * (k + 1)]
        acc8 = acc[0:8, :]
        for k in range(1, _BLK_R // 8):
            acc8 = acc8 + acc[8 * k:8 * (k + 1), :]
        return acc8

    an_ref[...] += fold(tf)
    ap_ref[...] += fold(sp_p * tf)
    al_ref[...] += fold(v)


def _run_tc_pass1(x0, x1, t):
    return pl.pallas_call(
        _tc_pass1,
        grid=(_GRID,),
        in_specs=[
            pl.BlockSpec((_BLK_R, _COLS), lambda i: (i, 0)),
            pl.BlockSpec((_BLK_R, _COLS), lambda i: (i, 0)),
            pl.BlockSpec((_BLK_R, _COLS), lambda i: (i, 0)),
        ],
        out_specs=[
            pl.BlockSpec((_BLK_R, _COLS), lambda i: (i, 0)),
            pl.BlockSpec((8, 128), lambda i: (0, 0)),
            pl.BlockSpec((8, 128), lambda i: (0, 0)),
            pl.BlockSpec((8, 128), lambda i: (0, 0)),
        ],
        out_shape=[
            jax.ShapeDtypeStruct((_ROWS, _COLS), jnp.float32),
            jax.ShapeDtypeStruct((8, 128), jnp.float32),
            jax.ShapeDtypeStruct((8, 128), jnp.float32),
            jax.ShapeDtypeStruct((8, 128), jnp.float32),
        ],
    )(x0, x1, t)


def _sc_hist1_body(v_hbm, cnt_hbm, buf0, buf1, cnt_v, sem0, sem1):
    wid = lax.axis_index("s") * 2 + lax.axis_index("c")
    base = wid * _PER_W

    def zero_body(i, _):
        cnt_v[pl.ds(i * _L, _L)] = jnp.zeros((_L,), jnp.float32)
        return 0

    lax.fori_loop(0, _NB1 // _L, zero_body, 0)

    ones = jnp.ones((_L,), jnp.float32)
    bufs = (buf0, buf1)
    sems = (sem0, sem1)
    copies = [None, None]
    copies[0] = pltpu.async_copy(v_hbm.at[pl.ds(base, _CH)], buf0, sem0)
    for c in range(_NCH):
        if c + 1 < _NCH:
            copies[(c + 1) % 2] = pltpu.async_copy(
                v_hbm.at[pl.ds(base + (c + 1) * _CH, _CH)],
                bufs[(c + 1) % 2], sems[(c + 1) % 2])
        copies[c % 2].wait()
        buf = bufs[c % 2]

        def vec_body(i, _):
            v = buf[pl.ds(i * _L, _L)]
            bits = lax.bitcast_convert_type(v, jnp.int32)
            b = lax.shift_right_logical(bits, 21)
            plsc.addupdate_scatter(cnt_v, [b], ones)
            return 0

        lax.fori_loop(0, _CH // _L, vec_body, 0, unroll=8)
    pltpu.sync_copy(cnt_v, cnt_hbm.at[wid])


def _run_sc_hist1(v):
    mesh = plsc.VectorSubcoreMesh(core_axis_name="c", subcore_axis_name="s")
    kern = functools.partial(
        pl.kernel,
        mesh=mesh,
        out_type=jax.ShapeDtypeStruct((_NW, _NB1), jnp.float32),
        scratch_types=[
            pltpu.VMEM((_CH,), jnp.float32),
            pltpu.VMEM((_CH,), jnp.float32),
            pltpu.VMEM((_NB1,), jnp.float32),
            pltpu.SemaphoreType.DMA,
            pltpu.SemaphoreType.DMA,
        ],
        compiler_params=pltpu.CompilerParams(needs_layout_passes=False),
    )(_sc_hist1_body)
    return kern(v.reshape(_B))


def _suffix_2d(h2d, rows, cols):
    ri = lax.broadcasted_iota(jnp.int32, (rows, rows), 0)
    rj = lax.broadcasted_iota(jnp.int32, (rows, rows), 1)
    m_rows = (ri < rj).astype(jnp.float32)
    ci = lax.broadcasted_iota(jnp.int32, (cols, cols), 0)
    cj = lax.broadcasted_iota(jnp.int32, (cols, cols), 1)
    m_cols = (ci > cj).astype(jnp.float32)
    rs = jnp.sum(h2d, axis=1, keepdims=True)
    rows_after = jax.lax.dot_general(
        m_rows, rs, (((1,), (0,)), ((), ())),
        preferred_element_type=jnp.float32,
        precision=lax.Precision.HIGHEST)
    within = jax.lax.dot_general(
        h2d, m_cols, (((1,), (0,)), ((), ())),
        preferred_element_type=jnp.float32,
        precision=lax.Precision.HIGHEST)
    return rows_after + within


def _select_bucket(cnt2d, rank, rows, cols):
    g = _suffix_2d(cnt2d, rows, cols)
    m = jnp.logical_and(g < rank, g + cnt2d >= rank).astype(jnp.float32)
    ri = lax.broadcasted_iota(jnp.int32, (rows, cols), 0)
    ci = lax.broadcasted_iota(jnp.int32, (rows, cols), 1)
    idx = jnp.sum((ri * cols + ci).astype(jnp.float32) * m)
    r_in = jnp.sum(m * (rank - g))
    return idx, r_in


def _tc_decide1(cnt_ref, an_ref, j_ref, r_ref):
    cnt2d = jnp.sum(cnt_ref[...], axis=0)
    n = jnp.sum(an_ref[...])
    idx, r_in = _select_bucket(cnt2d, n, 16, 128)
    j_ref[...] = jnp.full((8, 128), idx.astype(jnp.int32), jnp.int32)
    r_ref[...] = jnp.full((8, 128), r_in, jnp.float32)


def _run_tc_decide1(cnt1, an):
    return pl.pallas_call(
        _tc_decide1,
        out_shape=[
            jax.ShapeDtypeStruct((8, 128), jnp.int32),
            jax.ShapeDtypeStruct((8, 128), jnp.float32),
        ],
    )(cnt1.reshape(_NW, 16, 128), an)


def _sc_hist2_body(v_hbm, j_hbm, cnt_hbm, buf0, buf1, jbuf, cnt_v,
                   sem0, sem1):
    wid = lax.axis_index("s") * 2 + lax.axis_index("c")
    base = wid * _PER_W

    def zero_body(i, _):
        cnt_v[pl.ds(i * _L, _L)] = jnp.zeros((_L,), jnp.float32)
        return 0

    lax.fori_loop(0, _NB2 // _L, zero_body, 0)

    pltpu.sync_copy(j_hbm, jbuf)
    jv = jbuf[...]
    ones = jnp.ones((_L,), jnp.float32)
    bufs = (buf0, buf1)
    sems = (sem0, sem1)
    copies = [None, None]
    copies[0] = pltpu.async_copy(v_hbm.at[pl.ds(base, _CH)], buf0, sem0)
    for c in range(_NCH):
        if c + 1 < _NCH:
            copies[(c + 1) % 2] = pltpu.async_copy(
                v_hbm.at[pl.ds(base + (c + 1) * _CH, _CH)],
                bufs[(c + 1) % 2], sems[(c + 1) % 2])
        copies[c % 2].wait()
        buf = bufs[c % 2]

        def vec_body(i, _):
            v = buf[pl.ds(i * _L, _L)]
            bits = lax.bitcast_convert_type(v, jnp.int32)
            hi = lax.shift_right_logical(bits, 21)
            msk = hi == jv
            sub = jnp.bitwise_and(lax.shift_right_logical(bits, 9), _NB2 - 1)
            plsc.addupdate_scatter(cnt_v, [sub], ones, mask=msk)
            return 0

        lax.fori_loop(0, _CH // _L, vec_body, 0, unroll=8)
    pltpu.sync_copy(cnt_v, cnt_hbm.at[wid])


def _run_sc_hist2(v, jvec):
    mesh = plsc.VectorSubcoreMesh(core_axis_name="c", subcore_axis_name="s")
    kern = functools.partial(
        pl.kernel,
        mesh=mesh,
        out_type=jax.ShapeDtypeStruct((_NW, _NB2), jnp.float32),
        scratch_types=[
            pltpu.VMEM((_CH,), jnp.float32),
            pltpu.VMEM((_CH,), jnp.float32),
            pltpu.VMEM((_L,), jnp.int32),
            pltpu.VMEM((_NB2,), jnp.float32),
            pltpu.SemaphoreType.DMA,
            pltpu.SemaphoreType.DMA,
        ],
        compiler_params=pltpu.CompilerParams(needs_layout_passes=False),
    )(_sc_hist2_body)
    return kern(v.reshape(_B), jvec)


def _tc_decide2(cnt_ref, j_ref, r_ref, t_ref):
    cnt2d = jnp.sum(cnt_ref[...], axis=0)
    rank = r_ref[0, 0]
    idx, _ = _select_bucket(cnt2d, rank, 32, 128)
    jstar = j_ref[0, 0]
    tbits = jnp.bitwise_or(lax.shift_left(jstar, 21),
                           lax.shift_left(idx.astype(jnp.int32), 9))
    tval = lax.bitcast_convert_type(tbits, jnp.float32)
    t_ref[...] = jnp.full((8, 128), tval, jnp.float32)


def _run_tc_decide2(cnt2, jout, rout):
    return pl.pallas_call(
        _tc_decide2,
        out_shape=jax.ShapeDtypeStruct((8, 128), jnp.float32),
    )(cnt2.reshape(_NW, 32, 128), jout, rout)


def _tc_final(v_ref, t_ref, an_ref, ap_ref, al_ref, out_ref, cs_ref, cc_ref):
    @pl.when(pl.program_id(0) == 0)
    def _init():
        cs_ref[...] = jnp.zeros_like(cs_ref)
        cc_ref[...] = jnp.zeros_like(cc_ref)

    tval = t_ref[0, 0]
    v = v_ref[...]
    gt = (v > tval).astype(jnp.float32)

    def fold(z):
        acc = z[:, 0:128]
        for k in range(1, 8):
            acc = acc + z[:, 128 * k:128 * (k + 1)]
        acc8 = acc[0:8, :]
        for k in range(1, _BLK_R // 8):
            acc8 = acc8 + acc[8 * k:8 * (k + 1), :]
        return acc8

    cs_ref[...] += fold(v * gt)
    cc_ref[...] += fold(gt)

    @pl.when(pl.program_id(0) == _GRID - 1)
    def _finish():
        n = jnp.sum(an_ref[...])
        loss_pos = jnp.sum(ap_ref[...])
        loss_all_neg = jnp.sum(al_ref[...])
        sum_gt = jnp.sum(cs_ref[...])
        cnt_gt = jnp.sum(cc_ref[...])
        t_sum = sum_gt + (n - cnt_gt) * tval
        ce = (loss_pos + loss_all_neg) / jnp.float32(_B)
        res = (loss_pos + t_sum) / (2.0 * n) + ce
        out_ref[...] = jnp.full((1, 1), res, jnp.float32)


def _run_tc_final(v, tout, an, ap, al):
    return pl.pallas_call(
        _tc_final,
        grid=(_GRID,),
        in_specs=[
            pl.BlockSpec((_BLK_R, _COLS), lambda i: (i, 0)),
            pl.BlockSpec((8, 128), lambda i: (0, 0)),
            pl.BlockSpec((8, 128), lambda i: (0, 0)),
            pl.BlockSpec((8, 128), lambda i: (0, 0)),
            pl.BlockSpec((8, 128), lambda i: (0, 0)),
        ],
        out_specs=pl.BlockSpec((1, 1), lambda i: (0, 0)),
        out_shape=jax.ShapeDtypeStruct((1, 1), jnp.float32),
        scratch_shapes=[
            pltpu.VMEM((8, 128), jnp.float32),
            pltpu.VMEM((8, 128), jnp.float32),
        ],
    )(v, tout, an, ap, al)


def kernel(input, target):
    x0 = input[:, 0].reshape(_ROWS, _COLS)
    x1 = input[:, 1].reshape(_ROWS, _COLS)
    t = target.reshape(_ROWS, _COLS)
    v, an, ap, al = _run_tc_pass1(x0, x1, t)
    cnt1 = _run_sc_hist1(v)
    jout, rout = _run_tc_decide1(cnt1, an)
    jvec = jout[0, :_L].reshape(_L)
    cnt2 = _run_sc_hist2(v, jvec)
    tout = _run_tc_decide2(cnt2, jout, rout)
    out = _run_tc_final(v, tout, an, ap, al)
    return out.reshape(())
